# baseline (speedup 1.0000x reference)
import jax
import jax.numpy as jnp
from jax import lax
from jax.experimental import pallas as pl
from jax.experimental.pallas import tpu as pltpu
from jax.experimental.pallas import tpu_sc as plsc

B = 1024
S = 200
D = 64
NC = 2
NS = 16
NW = NC * NS
N = B * S
NSLAB = S * (B // 128)
PER_W = NSLAB // NW
TPW = N // NW


def _transpose_slab(rows_v, tbuf, p_v, s):
    iota = lax.iota(jnp.int32, 16)
    sbase = jnp.broadcast_to(s * D, (16,)).astype(jnp.int32)

    def dt_body(dt, _):
        for dr in range(8):
            d = dt * 8 + dr
            dvec = jnp.broadcast_to(d, (16,)).astype(jnp.int32)
            pval = plsc.load_gather(p_v, [sbase + dvec])
            for bc in range(8):
                jvec = iota + bc * 16
                vals = plsc.load_gather(rows_v, [jvec, dvec])
                tbuf[dt, dr, pl.ds(bc * 16, 16)] = vals + pval
        return 0

    lax.fori_loop(0, 8, dt_body, 0, unroll=False)


def _body(tok_hbm, e_hbm, p_hbm, out_hbm,
          idx_v, rows0, rows1, tb0, tb1, p_v,
          semG0, semG1, semW0, semW1):
    cid = lax.axis_index("c")
    sid = lax.axis_index("s")
    wid = sid * NC + cid
    k0 = wid * PER_W

    pltpu.sync_copy(tok_hbm.at[pl.ds(wid * TPW, TPW)], idx_v)
    pltpu.sync_copy(p_hbm, p_v)

    rows = [rows0, rows1]
    tbs = [tb0, tb1]
    semG = [semG0, semG1]
    semW = [semW0, semW1]

    def gather(i, b):
        return pltpu.async_copy(
            e_hbm.at[idx_v.at[pl.ds(i * 128, 128)]], rows[b], semG[b])

    def gather_wait(b):
        pltpu.make_async_copy(e_hbm.at[idx_v.at[pl.ds(0, 128)]],
                              rows[b], semG[b]).wait()

    def write_slab(i, b):
        k = k0 + i
        s = k >> 3
        bt = k & 7
        for dt in range(8):
            pltpu.async_copy(
                tbs[b].at[dt], out_hbm.at[(s * 8 + dt) * 8 + bt], semW[b])

    def write_wait(b):
        for dt in range(8):
            pltpu.make_async_copy(tbs[b].at[dt], out_hbm.at[dt],
                                  semW[b]).wait()

    gather(0, 0)

    def g_body(g, _):
        for b in range(2):
            i = g * 2 + b

            @pl.when(i + 1 < PER_W)
            def _():
                gather(i + 1, (b + 1) % 2)

            gather_wait(b)

            @pl.when(i >= 2)
            def _():
                write_wait(b)

            k = k0 + i
            _transpose_slab(rows[b], tbs[b], p_v, k >> 3)
            write_slab(i, b)
        return 0

    lax.fori_loop(0, PER_W // 2, g_body, 0, unroll=False)
    write_wait(0)
    write_wait(1)


def kernel(tokens, E, P):
    tok_t = tokens.T.reshape(N)
    e128 = jnp.pad(E, ((0, 0), (0, 64)))
    mesh = plsc.VectorSubcoreMesh(
        core_axis_name="c", subcore_axis_name="s", num_cores=NC, num_subcores=NS
    )
    run = pl.kernel(
        _body,
        out_type=jax.ShapeDtypeStruct((S * 8 * 8, 8, 128), jnp.float32),
        mesh=mesh,
        compiler_params=pltpu.CompilerParams(
            use_tc_tiling_on_sc=False, needs_layout_passes=False),
        scratch_types=[
            pltpu.VMEM((TPW,), jnp.int32),
            pltpu.VMEM((128, 128), jnp.float32),
            pltpu.VMEM((128, 128), jnp.float32),
            pltpu.VMEM((8, 8, 128), jnp.float32),
            pltpu.VMEM((8, 8, 128), jnp.float32),
            pltpu.VMEM((S * D,), jnp.float32),
            pltpu.SemaphoreType.DMA,
            pltpu.SemaphoreType.DMA,
            pltpu.SemaphoreType.DMA,
            pltpu.SemaphoreType.DMA,
        ],
    )
    out5 = run(tok_t, e128, P.reshape(S * D)).reshape(S, 8, 8, 8, 128)
    return out5.transpose(2, 4, 0, 1, 3).reshape(B, S, D)


# speedup vs baseline: 1.8759x; 1.8759x over previous
import jax
import jax.numpy as jnp
from jax import lax
from jax.experimental import pallas as pl
from jax.experimental.pallas import tpu as pltpu
from jax.experimental.pallas import tpu_sc as plsc

B = 1024
S = 200
D = 64
NC = 2
NS = 16
NW = NC * NS
N = B * S
PER_W = N // NW
CHUNK = 320
NCHUNK = PER_W // CHUNK


def _body(tok_hbm, pos_hbm, e_hbm, p_hbm, out_hbm,
          idx_v, pos_v, rows0, rows1, p_sh,
          semG0, semG1, semP, semS0, semS1):
    cid = lax.axis_index("c")
    sid = lax.axis_index("s")
    wid = sid * NC + cid
    base_w = wid * PER_W

    @pl.when(sid == 0)
    def _():
        pltpu.sync_copy(p_hbm, p_sh)

    plsc.subcore_barrier()

    pltpu.sync_copy(tok_hbm.at[pl.ds(base_w, PER_W)], idx_v)
    pltpu.sync_copy(pos_hbm.at[pl.ds(base_w, PER_W)], pos_v)

    rows = [rows0, rows1]
    semG = [semG0, semG1]
    semS = [semS0, semS1]

    def gather(c, b):
        def q_body(q, _):
            toks = idx_v[pl.ds(c * CHUNK + q * 16, 16)]
            for l in range(16):
                pltpu.async_copy(
                    e_hbm.at[toks[l]],
                    rows[b].at[q * 16 + l, pl.ds(0, D)], semG[b])
            return 0
        lax.fori_loop(0, CHUNK // 16, q_body, 0, unroll=False)

    def gather_wait(b):
        pltpu.make_async_copy(
            out_hbm.at[pl.ds(0, CHUNK // 2)],
            rows[b].at[pl.ds(0, CHUNK // 2)], semG[b]).wait()

    def padd(c, b):
        return pltpu.async_copy(
            p_sh.at[pos_v.at[pl.ds(c * CHUNK, CHUNK)]], rows[b],
            semP, add=True)

    def store(c, b):
        return pltpu.async_copy(
            rows[b], out_hbm.at[pl.ds(base_w + c * CHUNK, CHUNK)], semS[b])

    def store_wait(b):
        pltpu.make_async_copy(
            rows[b], out_hbm.at[pl.ds(0, CHUNK)], semS[b]).wait()

    gather(0, 0)

    def c_body(g, _):
        for b in range(2):
            c = g * 2 + b
            gather_wait(b)
            pa = padd(c, b)

            @pl.when(c + 1 < NCHUNK)
            def _():
                @pl.when(c >= 1)
                def _():
                    store_wait((b + 1) % 2)
                gather(c + 1, (b + 1) % 2)

            pa.wait()
            store(c, b)
        return 0

    lax.fori_loop(0, NCHUNK // 2, c_body, 0, unroll=False)
    store_wait(0)
    store_wait(1)


def kernel(tokens, E, P):
    p128 = jnp.pad(P, ((0, 0), (0, 64)))
    mesh = plsc.VectorSubcoreMesh(
        core_axis_name="c", subcore_axis_name="s", num_cores=NC, num_subcores=NS
    )
    run = pl.kernel(
        _body,
        out_type=jax.ShapeDtypeStruct((N, 128), jnp.float32),
        mesh=mesh,
        compiler_params=pltpu.CompilerParams(
            use_tc_tiling_on_sc=True, needs_layout_passes=False),
        scratch_types=[
            pltpu.VMEM((PER_W,), jnp.int32),
            pltpu.VMEM((PER_W,), jnp.int32),
            pltpu.VMEM((CHUNK, 128), jnp.float32),
            pltpu.VMEM((CHUNK, 128), jnp.float32),
            pltpu.VMEM_SHARED((S, 128), jnp.float32),
            pltpu.SemaphoreType.DMA,
            pltpu.SemaphoreType.DMA,
            pltpu.SemaphoreType.DMA,
            pltpu.SemaphoreType.DMA,
            pltpu.SemaphoreType.DMA,
        ],
    )
    pos = jnp.broadcast_to(jnp.arange(S, dtype=jnp.int32)[None, :], (B, S))
    out = run(tokens.reshape(N), pos.reshape(N), E, p128)
    return out[:, :D].reshape(B, S, D)
